# Initial kernel scaffold; baseline (speedup 1.0000x reference)
import jax
import jax.numpy as jnp
from jax.experimental import pallas as pl


def kernel(z, pos_embed):
    raise NotImplementedError("write your pallas kernel here")


# speedup vs baseline: 2.7232x; 2.7232x over previous
import jax
import jax.numpy as jnp
from jax.experimental import pallas as pl

SEQ_LEN = 2048
D_MODEL = 1024
N_REP = 4
BS = 256


def _body(emb_ref, out_ref):
    emb = emb_ref[...]
    out_ref[...] = jnp.broadcast_to(emb[:, None, :], (BS, N_REP, D_MODEL))


def kernel(z, pos_embed):
    out = pl.pallas_call(
        _body,
        grid=(SEQ_LEN // BS,),
        in_specs=[pl.BlockSpec((BS, D_MODEL), lambda i: (i, 0))],
        out_specs=pl.BlockSpec((BS, N_REP, D_MODEL), lambda i: (i, 0, 0)),
        out_shape=jax.ShapeDtypeStruct((SEQ_LEN, N_REP, D_MODEL), z.dtype),
    )(pos_embed)
    return out


# speedup vs baseline: 2.8276x; 1.0384x over previous
import jax
import jax.numpy as jnp
from jax.experimental import pallas as pl

SEQ_LEN = 2048
D_MODEL = 1024
N_REP = 4
BS = 512


def _body(emb_ref, out_ref):
    emb = emb_ref[...]
    out_ref[...] = jnp.broadcast_to(emb[:, None, :], (BS, N_REP, D_MODEL))


def kernel(z, pos_embed):
    out = pl.pallas_call(
        _body,
        grid=(SEQ_LEN // BS,),
        in_specs=[pl.BlockSpec((BS, D_MODEL), lambda i: (i, 0))],
        out_specs=pl.BlockSpec((BS, N_REP, D_MODEL), lambda i: (i, 0, 0)),
        out_shape=jax.ShapeDtypeStruct((SEQ_LEN, N_REP, D_MODEL), z.dtype),
    )(pos_embed)
    return out
